# baseline (speedup 1.0000x reference)
import functools

import jax
import jax.numpy as jnp
from jax import lax
from jax.experimental import pallas as pl
from jax.experimental.pallas import tpu as pltpu
from jax.experimental.pallas import tpu_sc as plsc

N_NODES = 10000
D = 128
E = 320000
BATCH = 4096

NC, NS = 2, 16
NW = NC * NS
EPW = E // NW
K = 128
BPW = BATCH // NW
BPT = BATCH // NS
GARBAGE = BATCH
ACC = 4112
ZR = ACC // NS
VL = 16
NMARK = N_NODES + VL


def _tc1_body(x_ref, w_ref, b_ref, a_ref, bb_ref):
    x = x_ref[...]
    a_ref[...] = jnp.dot(x, w_ref[:D, :], preferred_element_type=jnp.float32)
    bb_ref[...] = (
        jnp.dot(x, w_ref[D:, :], preferred_element_type=jnp.float32) + b_ref[...]
    )


def _precompute_ab(x, W1, b1):
    blk = N_NODES // 10
    return pl.pallas_call(
        _tc1_body,
        grid=(N_NODES // blk,),
        in_specs=[
            pl.BlockSpec((blk, D), lambda i: (i, 0)),
            pl.BlockSpec((2 * D, D), lambda i: (0, 0)),
            pl.BlockSpec((1, D), lambda i: (0, 0)),
        ],
        out_specs=[
            pl.BlockSpec((blk, D), lambda i: (i, 0)),
            pl.BlockSpec((blk, D), lambda i: (i, 0)),
        ],
        out_shape=[
            jax.ShapeDtypeStruct((N_NODES, D), jnp.float32),
            jax.ShapeDtypeStruct((N_NODES, D), jnp.float32),
        ],
    )(x, W1, b1.reshape(1, D))


def _sc_body(src_hbm, dst_hbm, emb_hbm, a_hbm, b_hbm, batch_hbm,
             xb_out, aggb_out,
             srcv0, gdstv0, cidv0, arows0, brows0,
             srcv1, gdstv1, cidv1, arows1, brows1,
             bidxv, markv, srcsh, dstsh,
             aggsh, sa0, sb0, sa1, sb1):
    cid = lax.axis_index("c")
    sid = lax.axis_index("s")
    wid = sid * NC + cid

    zero16 = jnp.zeros((VL,), jnp.float32)
    ones16 = jnp.ones((VL,), jnp.int32)

    def zrow(i, carry):
        for j in range(D // VL):
            arows0[i, pl.ds(j * VL, VL)] = zero16
        return carry

    lax.fori_loop(0, K, zrow, None)
    z0 = sid * ZR
    pltpu.sync_copy(arows0, aggsh.at[pl.ds(z0, K)])
    pltpu.sync_copy(arows0, aggsh.at[pl.ds(z0 + K, K)])
    pltpu.sync_copy(arows0.at[pl.ds(0, ZR - 2 * K)],
                    aggsh.at[pl.ds(z0 + 2 * K, ZR - 2 * K)])

    def zmark(i, carry):
        markv[pl.ds(i * VL, VL)] = jnp.zeros((VL,), jnp.int32)
        return carry

    lax.fori_loop(0, NMARK // VL, zmark, None)

    for ch in range(BATCH // BPW):
        pltpu.sync_copy(batch_hbm.at[pl.ds(ch * BPW, BPW)], bidxv)

        def scat(i, carry):
            idx = bidxv[pl.ds(i * VL, VL)]
            plsc.store_scatter(markv, [idx], ones16)
            return carry

        lax.fori_loop(0, BPW // VL, scat, None)

    def scan(i, carry):
        s = pl.ds(i * VL, VL)
        f = markv[s]
        ids = carry + plsc.cumsum(f) - 1
        markv[s] = jnp.where(f > 0, ids, jnp.full((VL,), GARBAGE, jnp.int32))
        return carry + jnp.sum(f)

    lax.fori_loop(0, NMARK // VL, scan, jnp.int32(0))

    pltpu.sync_copy(src_hbm.at[pl.ds(wid * EPW, EPW)], srcsh.at[pl.ds(0, EPW)])
    pltpu.sync_copy(dst_hbm.at[pl.ds(wid * EPW, EPW)], dstsh.at[pl.ds(0, EPW)])

    def compress(i, cur):
        s = pl.ds(i * VL, VL)
        d = dstsh[s]
        sv = srcsh[s]
        keep = plsc.load_gather(markv, [d]) != GARBAGE
        plsc.store_compressed(dstsh.at[pl.ds(cur, VL)], d, mask=keep)
        plsc.store_compressed(srcsh.at[pl.ds(cur, VL)], sv, mask=keep)
        return cur + jnp.sum(keep.astype(jnp.int32))

    cnt = lax.fori_loop(0, EPW // VL, compress, jnp.int32(0))

    pad_dst = jnp.full((VL,), N_NODES, jnp.int32)
    pad_src = jnp.zeros((VL,), jnp.int32)

    def padk(i, carry):
        dstsh[pl.ds(cnt + i * VL, VL)] = pad_dst
        srcsh[pl.ds(cnt + i * VL, VL)] = pad_src
        return carry

    lax.fori_loop(0, K // VL, padk, None)
    nchunks = (cnt + K - 1) >> 7

    plsc.subcore_barrier()

    sets = ((srcv0, gdstv0, cidv0, arows0, brows0, sa0, sb0),
            (srcv1, gdstv1, cidv1, arows1, brows1, sa1, sb1))

    def build_issue(cc, p):
        sv, gv, cv, ar, br, sa, sb = sets[p]

        @pl.when(cc >= 2)
        def _():
            pltpu.make_async_copy(ar, aggsh.at[cv], sb).wait()

        for j in range(K // VL):
            s = pl.ds(j * VL, VL)
            dv = dstsh[pl.ds(cc * K + j * VL, VL)]
            sv[s] = srcsh[pl.ds(cc * K + j * VL, VL)]
            gv[s] = jnp.minimum(dv, N_NODES - 1)
            cv[s] = plsc.load_gather(markv, [dv])
        pltpu.async_copy(a_hbm.at[sv], ar, sa)
        pltpu.async_copy(b_hbm.at[gv], br, sa)

    def step(c, p):
        sv, gv, cv, ar, br, sa, sb = sets[p]
        pltpu.make_async_copy(a_hbm.at[sv], ar, sa).wait()
        pltpu.make_async_copy(b_hbm.at[gv], br, sa).wait()

        @pl.when(c + 1 < nchunks)
        def _():
            build_issue(c + 1, 1 - p)

        def fuse(i, inner):
            for j in range(D // VL):
                s = pl.ds(j * VL, VL)
                ar[i, s] = jnp.maximum(ar[i, s] + br[i, s], 0.0)
            return inner

        lax.fori_loop(0, K, fuse, None)
        pltpu.async_copy(ar, aggsh.at[cv], sb, add=True)

    @pl.when(nchunks > 0)
    def _():
        build_issue(0, 0)

    def pair(c2, carry):
        c = c2 * 2

        @pl.when(c < nchunks)
        def _():
            step(c, 0)

        @pl.when(c + 1 < nchunks)
        def _():
            step(c + 1, 1)

        return carry

    lax.fori_loop(0, (nchunks + 1) >> 1, pair, None)

    @pl.when(nchunks >= 1)
    def _():
        pltpu.make_async_copy(arows0, aggsh.at[cidv0], sb0).wait()

    @pl.when(nchunks >= 2)
    def _():
        pltpu.make_async_copy(arows1, aggsh.at[cidv1], sb1).wait()

    plsc.subcore_barrier()

    base = wid * BPW
    pltpu.sync_copy(batch_hbm.at[pl.ds(base, BPW)], bidxv)
    pltpu.async_copy(emb_hbm.at[bidxv], arows0, sa0).wait()
    pltpu.sync_copy(arows0, xb_out.at[pl.ds(base, BPW)])

    for r in range(BPT // BPW):
        b0 = sid * BPT + r * BPW
        pltpu.sync_copy(batch_hbm.at[pl.ds(b0, BPW)], bidxv)
        for j in range(BPW // VL):
            s = pl.ds(j * VL, VL)
            bidxv[s] = plsc.load_gather(markv, [bidxv[s]])
        pltpu.async_copy(aggsh.at[bidxv], arows0, sb0).wait()
        pltpu.sync_copy(arows0, aggb_out.at[cid, pl.ds(b0, BPW)])


_sc_call = pl.kernel(
    _sc_body,
    out_type=(
        jax.ShapeDtypeStruct((BATCH, D), jnp.float32),
        jax.ShapeDtypeStruct((NC, BATCH, D), jnp.float32),
    ),
    mesh=plsc.VectorSubcoreMesh(core_axis_name="c", subcore_axis_name="s"),
    scratch_types=[
        pltpu.VMEM((K,), jnp.int32),
        pltpu.VMEM((K,), jnp.int32),
        pltpu.VMEM((K,), jnp.int32),
        pltpu.VMEM((K, D), jnp.float32),
        pltpu.VMEM((K, D), jnp.float32),
        pltpu.VMEM((K,), jnp.int32),
        pltpu.VMEM((K,), jnp.int32),
        pltpu.VMEM((K,), jnp.int32),
        pltpu.VMEM((K, D), jnp.float32),
        pltpu.VMEM((K, D), jnp.float32),
        pltpu.VMEM((BPW,), jnp.int32),
        pltpu.VMEM((NMARK,), jnp.int32),
        pltpu.VMEM((EPW + K,), jnp.int32),
        pltpu.VMEM((EPW + K,), jnp.int32),
        pltpu.VMEM_SHARED((ACC, D), jnp.float32),
        pltpu.SemaphoreType.DMA,
        pltpu.SemaphoreType.DMA,
        pltpu.SemaphoreType.DMA,
        pltpu.SemaphoreType.DMA,
    ],
    compiler_params=pltpu.CompilerParams(needs_layout_passes=False),
)


def _tc2_body(xb_ref, a0_ref, a1_ref, w_ref, b_ref, o_ref):
    o_ref[...] = (
        jnp.dot(xb_ref[...], w_ref[:D, :], preferred_element_type=jnp.float32)
        + jnp.dot(a0_ref[...] + a1_ref[...], w_ref[D:, :],
                  preferred_element_type=jnp.float32)
        + b_ref[...]
    )


def _final(xb, a0, a1, W2, b2):
    blk = 1024
    return pl.pallas_call(
        _tc2_body,
        grid=(BATCH // blk,),
        in_specs=[
            pl.BlockSpec((blk, D), lambda i: (i, 0)),
            pl.BlockSpec((blk, D), lambda i: (i, 0)),
            pl.BlockSpec((blk, D), lambda i: (i, 0)),
            pl.BlockSpec((2 * D, D), lambda i: (0, 0)),
            pl.BlockSpec((1, D), lambda i: (0, 0)),
        ],
        out_specs=pl.BlockSpec((blk, D), lambda i: (i, 0)),
        out_shape=jax.ShapeDtypeStruct((BATCH, D), jnp.float32),
    )(xb, a0, a1, W2, b2.reshape(1, D))


def kernel(batch, edge_index, emb, W1, b1, W2, b2):
    a, bb = _precompute_ab(emb, W1, b1)
    xb, aggb = _sc_call(edge_index[0], edge_index[1], emb, a, bb, batch)
    return _final(xb, aggb[0], aggb[1], W2, b2)


# speedup vs baseline: 1.3590x; 1.3590x over previous
import functools

import jax
import jax.numpy as jnp
from jax import lax
from jax.experimental import pallas as pl
from jax.experimental.pallas import tpu as pltpu
from jax.experimental.pallas import tpu_sc as plsc

N_NODES = 10000
D = 128
E = 320000
BATCH = 4096

NC, NS = 2, 16
NW = NC * NS
EPW = E // NW
CK = 64
CKS = 6
NSETS = 4
BPW = BATCH // NW
BPT = BATCH // NS
GARBAGE = BATCH
ACC = 4112
ZR = ACC // NS
VL = 16
NMARK = N_NODES + VL


def _tc1_body(x_ref, w_ref, b_ref, a_ref, bb_ref):
    x = x_ref[...]
    a_ref[...] = jnp.dot(x, w_ref[:D, :], preferred_element_type=jnp.float32)
    bb_ref[...] = (
        jnp.dot(x, w_ref[D:, :], preferred_element_type=jnp.float32) + b_ref[...]
    )


def _precompute_ab(x, W1, b1):
    blk = N_NODES // 10
    return pl.pallas_call(
        _tc1_body,
        grid=(N_NODES // blk,),
        in_specs=[
            pl.BlockSpec((blk, D), lambda i: (i, 0)),
            pl.BlockSpec((2 * D, D), lambda i: (0, 0)),
            pl.BlockSpec((1, D), lambda i: (0, 0)),
        ],
        out_specs=[
            pl.BlockSpec((blk, D), lambda i: (i, 0)),
            pl.BlockSpec((blk, D), lambda i: (i, 0)),
        ],
        out_shape=[
            jax.ShapeDtypeStruct((N_NODES, D), jnp.float32),
            jax.ShapeDtypeStruct((N_NODES, D), jnp.float32),
        ],
    )(x, W1, b1.reshape(1, D))


def _sc_body(src_hbm, dst_hbm, emb_hbm, a_hbm, b_hbm, batch_hbm,
             xb_out, aggb_out,
             srcv0, gdstv0, cidv0, arows0, brows0,
             srcv1, gdstv1, cidv1, arows1, brows1,
             srcv2, gdstv2, cidv2, arows2, brows2,
             srcv3, gdstv3, cidv3, arows3, brows3,
             bidxv, markv, srcsh, dstsh,
             aggsh, sg0, sg1, sg2, sg3):
    cid = lax.axis_index("c")
    sid = lax.axis_index("s")
    wid = sid * NC + cid

    zero16 = jnp.zeros((VL,), jnp.float32)
    ones16 = jnp.ones((VL,), jnp.int32)

    def zrow(i, carry):
        for j in range(D // VL):
            arows0[i, pl.ds(j * VL, VL)] = zero16
        return carry

    lax.fori_loop(0, CK, zrow, None)
    z0 = sid * ZR
    for r in range(ZR // CK):
        pltpu.sync_copy(arows0, aggsh.at[pl.ds(z0 + r * CK, CK)])
    pltpu.sync_copy(arows0.at[pl.ds(0, ZR - (ZR // CK) * CK)],
                    aggsh.at[pl.ds(z0 + (ZR // CK) * CK, ZR - (ZR // CK) * CK)])

    def zmark(i, carry):
        markv[pl.ds(i * VL, VL)] = jnp.zeros((VL,), jnp.int32)
        return carry

    lax.fori_loop(0, NMARK // VL, zmark, None)

    for ch in range(BATCH // BPW):
        pltpu.sync_copy(batch_hbm.at[pl.ds(ch * BPW, BPW)], bidxv)

        def scat(i, carry):
            idx = bidxv[pl.ds(i * VL, VL)]
            plsc.store_scatter(markv, [idx], ones16)
            return carry

        lax.fori_loop(0, BPW // VL, scat, None)

    def scan(i, carry):
        s = pl.ds(i * VL, VL)
        f = markv[s]
        ids = carry + plsc.cumsum(f) - 1
        markv[s] = jnp.where(f > 0, ids, jnp.full((VL,), GARBAGE, jnp.int32))
        return carry + jnp.sum(f)

    lax.fori_loop(0, NMARK // VL, scan, jnp.int32(0))

    pltpu.sync_copy(src_hbm.at[pl.ds(wid * EPW, EPW)], srcsh.at[pl.ds(0, EPW)])
    pltpu.sync_copy(dst_hbm.at[pl.ds(wid * EPW, EPW)], dstsh.at[pl.ds(0, EPW)])

    def compress(i, cur):
        s = pl.ds(i * VL, VL)
        d = dstsh[s]
        sv = srcsh[s]
        keep = plsc.load_gather(markv, [d]) != GARBAGE
        plsc.store_compressed(dstsh.at[pl.ds(cur, VL)], d, mask=keep)
        plsc.store_compressed(srcsh.at[pl.ds(cur, VL)], sv, mask=keep)
        return cur + jnp.sum(keep.astype(jnp.int32))

    cnt = lax.fori_loop(0, EPW // VL, compress, jnp.int32(0))

    pad_dst = jnp.full((VL,), N_NODES, jnp.int32)
    pad_src = jnp.zeros((VL,), jnp.int32)

    def padk(i, carry):
        dstsh[pl.ds(cnt + i * VL, VL)] = pad_dst
        srcsh[pl.ds(cnt + i * VL, VL)] = pad_src
        return carry

    lax.fori_loop(0, CK // VL, padk, None)
    nchunks = (cnt + CK - 1) >> CKS

    plsc.subcore_barrier()

    sets = ((srcv0, gdstv0, cidv0, arows0, brows0, sg0),
            (srcv1, gdstv1, cidv1, arows1, brows1, sg1),
            (srcv2, gdstv2, cidv2, arows2, brows2, sg2),
            (srcv3, gdstv3, cidv3, arows3, brows3, sg3))

    def build_issue(cc, p):
        sv, gv, cv, ar, br, sg = sets[p]
        for j in range(CK // VL):
            s = pl.ds(j * VL, VL)
            dv = dstsh[pl.ds(cc * CK + j * VL, VL)]
            sv[s] = srcsh[pl.ds(cc * CK + j * VL, VL)]
            gv[s] = jnp.minimum(dv, N_NODES - 1)
            cv[s] = plsc.load_gather(markv, [dv])
        pltpu.async_copy(a_hbm.at[sv], ar, sg)
        pltpu.async_copy(b_hbm.at[gv], br, sg)

    def step(c, p):
        sv, gv, cv, ar, br, sg = sets[p]
        pltpu.make_async_copy(a_hbm.at[sv], ar, sg).wait()
        pltpu.make_async_copy(b_hbm.at[gv], br, sg).wait()

        @pl.when(c + NSETS - 1 < nchunks)
        def _():
            build_issue(c + NSETS - 1, (p + NSETS - 1) % NSETS)

        def fuse(i, inner):
            for j in range(D // VL):
                s = pl.ds(j * VL, VL)
                ar[i, s] = jnp.maximum(ar[i, s] + br[i, s], 0.0)
            return inner

        lax.fori_loop(0, CK, fuse, None)
        pltpu.sync_copy(ar, aggsh.at[cv], add=True)

    for i in range(NSETS - 1):
        @pl.when(i < nchunks)
        def _(i=i):
            build_issue(i, i)

    def quad(c4, carry):
        c = c4 * NSETS
        for b in range(NSETS):
            @pl.when(c + b < nchunks)
            def _(b=b):
                step(c + b, b)
        return carry

    lax.fori_loop(0, (nchunks + NSETS - 1) >> 2, quad, None)
    plsc.subcore_barrier()

    base = wid * BPW
    pltpu.sync_copy(batch_hbm.at[pl.ds(base, BPW)], bidxv)
    for h in range(BPW // CK):
        pltpu.async_copy(emb_hbm.at[bidxv.at[pl.ds(h * CK, CK)]], arows0, sg0).wait()
        pltpu.sync_copy(arows0, xb_out.at[pl.ds(base + h * CK, CK)])

    for r in range(BPT // BPW):
        b0 = sid * BPT + r * BPW
        pltpu.sync_copy(batch_hbm.at[pl.ds(b0, BPW)], bidxv)
        for j in range(BPW // VL):
            s = pl.ds(j * VL, VL)
            bidxv[s] = plsc.load_gather(markv, [bidxv[s]])
        for h in range(BPW // CK):
            pltpu.async_copy(aggsh.at[bidxv.at[pl.ds(h * CK, CK)]], arows1, sg1).wait()
            pltpu.sync_copy(arows1, aggb_out.at[cid, pl.ds(b0 + h * CK, CK)])


def _set_scratch():
    return [
        pltpu.VMEM((CK,), jnp.int32),
        pltpu.VMEM((CK,), jnp.int32),
        pltpu.VMEM((CK,), jnp.int32),
        pltpu.VMEM((CK, D), jnp.float32),
        pltpu.VMEM((CK, D), jnp.float32),
    ]


_sc_call = pl.kernel(
    _sc_body,
    out_type=(
        jax.ShapeDtypeStruct((BATCH, D), jnp.float32),
        jax.ShapeDtypeStruct((NC, BATCH, D), jnp.float32),
    ),
    mesh=plsc.VectorSubcoreMesh(core_axis_name="c", subcore_axis_name="s"),
    scratch_types=(
        _set_scratch() + _set_scratch() + _set_scratch() + _set_scratch() + [
            pltpu.VMEM((BPW,), jnp.int32),
            pltpu.VMEM((NMARK,), jnp.int32),
            pltpu.VMEM((EPW + CK,), jnp.int32),
            pltpu.VMEM((EPW + CK,), jnp.int32),
            pltpu.VMEM_SHARED((ACC, D), jnp.float32),
            pltpu.SemaphoreType.DMA,
            pltpu.SemaphoreType.DMA,
            pltpu.SemaphoreType.DMA,
            pltpu.SemaphoreType.DMA,
        ]
    ),
    compiler_params=pltpu.CompilerParams(needs_layout_passes=False),
)


def _tc2_body(xb_ref, a0_ref, a1_ref, w_ref, b_ref, o_ref):
    o_ref[...] = (
        jnp.dot(xb_ref[...], w_ref[:D, :], preferred_element_type=jnp.float32)
        + jnp.dot(a0_ref[...] + a1_ref[...], w_ref[D:, :],
                  preferred_element_type=jnp.float32)
        + b_ref[...]
    )


def _final(xb, a0, a1, W2, b2):
    blk = 1024
    return pl.pallas_call(
        _tc2_body,
        grid=(BATCH // blk,),
        in_specs=[
            pl.BlockSpec((blk, D), lambda i: (i, 0)),
            pl.BlockSpec((blk, D), lambda i: (i, 0)),
            pl.BlockSpec((blk, D), lambda i: (i, 0)),
            pl.BlockSpec((2 * D, D), lambda i: (0, 0)),
            pl.BlockSpec((1, D), lambda i: (0, 0)),
        ],
        out_specs=pl.BlockSpec((blk, D), lambda i: (i, 0)),
        out_shape=jax.ShapeDtypeStruct((BATCH, D), jnp.float32),
    )(xb, a0, a1, W2, b2.reshape(1, D))


def kernel(batch, edge_index, emb, W1, b1, W2, b2):
    a, bb = _precompute_ab(emb, W1, b1)
    xb, aggb = _sc_call(edge_index[0], edge_index[1], emb, a, bb, batch)
    return _final(xb, aggb[0], aggb[1], W2, b2)


# speedup vs baseline: 1.4263x; 1.0495x over previous
import functools

import jax
import jax.numpy as jnp
from jax import lax
from jax.experimental import pallas as pl
from jax.experimental.pallas import tpu as pltpu
from jax.experimental.pallas import tpu_sc as plsc

N_NODES = 10000
D = 128
E = 320000
BATCH = 4096

NC, NS = 2, 16
NW = NC * NS
EPW = E // NW
CK = 32
CKS = 5
NSETS = 8
NSETS_S = 3
SEGS = (5008, 4992)
BPW = BATCH // NW
BPT = BATCH // NS
GARBAGE = BATCH
ACC = 4112
ZR = ACC // NS
VL = 16
NMARK = N_NODES + VL


def _tc1_body(x_ref, w_ref, b_ref, a_ref, bb_ref):
    x = x_ref[...]
    a_ref[...] = jnp.dot(x, w_ref[:D, :], preferred_element_type=jnp.float32)
    bb_ref[...] = (
        jnp.dot(x, w_ref[D:, :], preferred_element_type=jnp.float32) + b_ref[...]
    )


def _precompute_ab(x, W1, b1):
    blk = N_NODES // 10
    return pl.pallas_call(
        _tc1_body,
        grid=(N_NODES // blk,),
        in_specs=[
            pl.BlockSpec((blk, D), lambda i: (i, 0)),
            pl.BlockSpec((2 * D, D), lambda i: (0, 0)),
            pl.BlockSpec((1, D), lambda i: (0, 0)),
        ],
        out_specs=[
            pl.BlockSpec((blk, D), lambda i: (i, 0)),
            pl.BlockSpec((blk, D), lambda i: (i, 0)),
        ],
        out_shape=[
            jax.ShapeDtypeStruct((N_NODES, D), jnp.float32),
            jax.ShapeDtypeStruct((N_NODES, D), jnp.float32),
        ],
    )(x, W1, b1.reshape(1, D))


def _sc_body(src_hbm, dst_hbm, emb_hbm, a_hbm, b_hbm, batch_hbm,
             xb_out, aggb_out, *scr):
    sets = tuple(
        tuple(scr[i * 5:(i + 1) * 5]) + (scr[5 * NSETS + 5 + i],)
        for i in range(NSETS)
    )
    bidxv, markv, srcsh, dstsh, aggsh = scr[5 * NSETS:5 * NSETS + 5]
    arows0 = sets[0][3]
    arows1 = sets[1][3]
    sg0 = sets[0][5]
    sg1 = sets[1][5]
    cid = lax.axis_index("c")
    sid = lax.axis_index("s")
    wid = sid * NC + cid

    zero16 = jnp.zeros((VL,), jnp.float32)
    ones16 = jnp.ones((VL,), jnp.int32)

    def zrow(i, carry):
        for j in range(D // VL):
            arows0[i, pl.ds(j * VL, VL)] = zero16
        return carry

    lax.fori_loop(0, CK, zrow, None)
    z0 = sid * ZR
    for r in range(ZR // CK):
        pltpu.sync_copy(arows0, aggsh.at[pl.ds(z0 + r * CK, CK)])
    pltpu.sync_copy(arows0.at[pl.ds(0, ZR - (ZR // CK) * CK)],
                    aggsh.at[pl.ds(z0 + (ZR // CK) * CK, ZR - (ZR // CK) * CK)])

    def zmark(i, carry):
        markv[pl.ds(i * VL, VL)] = jnp.zeros((VL,), jnp.int32)
        return carry

    lax.fori_loop(0, NMARK // VL, zmark, None)

    for ch in range(BATCH // BPW):
        pltpu.sync_copy(batch_hbm.at[pl.ds(ch * BPW, BPW)], bidxv)

        def scat(i, carry):
            idx = bidxv[pl.ds(i * VL, VL)]
            plsc.store_scatter(markv, [idx], ones16)
            return carry

        lax.fori_loop(0, BPW // VL, scat, None)

    def scan(i, carry):
        s = pl.ds(i * VL, VL)
        f = markv[s]
        ids = carry + plsc.cumsum(f) - 1
        markv[s] = jnp.where(f > 0, ids, jnp.full((VL,), GARBAGE, jnp.int32))
        return carry + jnp.sum(f)

    lax.fori_loop(0, NMARK // VL, scan, jnp.int32(0))

    plsc.subcore_barrier()

    pad_dst = jnp.full((VL,), N_NODES, jnp.int32)
    pad_src = jnp.zeros((VL,), jnp.int32)

    def run_segment(s0, selen):
        e0 = wid * EPW + s0
        pltpu.sync_copy(src_hbm.at[pl.ds(e0, selen)], srcsh.at[pl.ds(0, selen)])
        pltpu.sync_copy(dst_hbm.at[pl.ds(e0, selen)], dstsh.at[pl.ds(0, selen)])

        def compress(i, cur):
            s = pl.ds(i * VL, VL)
            d = dstsh[s]
            sv = srcsh[s]
            keep = plsc.load_gather(markv, [d]) != GARBAGE
            plsc.store_compressed(dstsh.at[pl.ds(cur, VL)], d, mask=keep)
            plsc.store_compressed(srcsh.at[pl.ds(cur, VL)], sv, mask=keep)
            return cur + jnp.sum(keep.astype(jnp.int32))

        cnt = lax.fori_loop(0, selen // VL, compress, jnp.int32(0))

        def padk(i, carry):
            dstsh[pl.ds(cnt + i * VL, VL)] = pad_dst
            srcsh[pl.ds(cnt + i * VL, VL)] = pad_src
            return carry

        lax.fori_loop(0, CK // VL, padk, None)
        nchunks = (cnt + CK - 1) >> CKS

        def build_issue(cc, p):
            sv, gv, cv, ar, br, sg = sets[p]
            for j in range(CK // VL):
                s = pl.ds(j * VL, VL)
                dv = dstsh[pl.ds(cc * CK + j * VL, VL)]
                sv[s] = srcsh[pl.ds(cc * CK + j * VL, VL)]
                gv[s] = jnp.minimum(dv, N_NODES - 1)
                cv[s] = plsc.load_gather(markv, [dv])
            pltpu.async_copy(a_hbm.at[sv], ar, sg)
            pltpu.async_copy(b_hbm.at[gv], br, sg)

        def step(c, p):
            sv, gv, cv, ar, br, sg = sets[p]
            pltpu.make_async_copy(a_hbm.at[sv], ar, sg).wait()
            pltpu.make_async_copy(b_hbm.at[gv], br, sg).wait()

            @pl.when(c + NSETS - 1 < nchunks)
            def _():
                build_issue(c + NSETS - 1, (p + NSETS - 1) % NSETS)

            def fuse(i, inner):
                for j in range(D // VL):
                    s = pl.ds(j * VL, VL)
                    ar[i, s] = jnp.maximum(ar[i, s] + br[i, s], 0.0)
                return inner

            lax.fori_loop(0, CK, fuse, None)
            pltpu.sync_copy(ar, aggsh.at[cv], add=True)

        for i in range(NSETS - 1):
            @pl.when(i < nchunks)
            def _(i=i):
                build_issue(i, i)

        def grp(cg, carry):
            c = cg * NSETS
            for b in range(NSETS):
                @pl.when(c + b < nchunks)
                def _(b=b):
                    step(c + b, b)
            return carry

        lax.fori_loop(0, (nchunks + NSETS - 1) >> NSETS_S, grp, None)

    s0 = 0
    for selen in SEGS:
        run_segment(s0, selen)
        s0 += selen
    plsc.subcore_barrier()

    base = wid * BPW
    pltpu.sync_copy(batch_hbm.at[pl.ds(base, BPW)], bidxv)
    for h in range(BPW // CK):
        pltpu.async_copy(emb_hbm.at[bidxv.at[pl.ds(h * CK, CK)]], arows0, sg0).wait()
        pltpu.sync_copy(arows0, xb_out.at[pl.ds(base + h * CK, CK)])

    for r in range(BPT // BPW):
        b0 = sid * BPT + r * BPW
        pltpu.sync_copy(batch_hbm.at[pl.ds(b0, BPW)], bidxv)
        for j in range(BPW // VL):
            s = pl.ds(j * VL, VL)
            bidxv[s] = plsc.load_gather(markv, [bidxv[s]])
        for h in range(BPW // CK):
            pltpu.async_copy(aggsh.at[bidxv.at[pl.ds(h * CK, CK)]], arows1, sg1).wait()
            pltpu.sync_copy(arows1, aggb_out.at[cid, pl.ds(b0 + h * CK, CK)])


def _set_scratch():
    return [
        pltpu.VMEM((CK,), jnp.int32),
        pltpu.VMEM((CK,), jnp.int32),
        pltpu.VMEM((CK,), jnp.int32),
        pltpu.VMEM((CK, D), jnp.float32),
        pltpu.VMEM((CK, D), jnp.float32),
    ]


_sc_call = pl.kernel(
    _sc_body,
    out_type=(
        jax.ShapeDtypeStruct((BATCH, D), jnp.float32),
        jax.ShapeDtypeStruct((NC, BATCH, D), jnp.float32),
    ),
    mesh=plsc.VectorSubcoreMesh(core_axis_name="c", subcore_axis_name="s"),
    scratch_types=(
        sum((_set_scratch() for _ in range(NSETS)), []) + [
            pltpu.VMEM((BPW,), jnp.int32),
            pltpu.VMEM((NMARK,), jnp.int32),
            pltpu.VMEM((SEGS[0] + CK,), jnp.int32),
            pltpu.VMEM((SEGS[0] + CK,), jnp.int32),
            pltpu.VMEM_SHARED((ACC, D), jnp.float32),
        ] + [pltpu.SemaphoreType.DMA] * NSETS
    ),
    compiler_params=pltpu.CompilerParams(needs_layout_passes=False),
)


def _tc2_body(xb_ref, a0_ref, a1_ref, w_ref, b_ref, o_ref):
    o_ref[...] = (
        jnp.dot(xb_ref[...], w_ref[:D, :], preferred_element_type=jnp.float32)
        + jnp.dot(a0_ref[...] + a1_ref[...], w_ref[D:, :],
                  preferred_element_type=jnp.float32)
        + b_ref[...]
    )


def _final(xb, a0, a1, W2, b2):
    blk = 1024
    return pl.pallas_call(
        _tc2_body,
        grid=(BATCH // blk,),
        in_specs=[
            pl.BlockSpec((blk, D), lambda i: (i, 0)),
            pl.BlockSpec((blk, D), lambda i: (i, 0)),
            pl.BlockSpec((blk, D), lambda i: (i, 0)),
            pl.BlockSpec((2 * D, D), lambda i: (0, 0)),
            pl.BlockSpec((1, D), lambda i: (0, 0)),
        ],
        out_specs=pl.BlockSpec((blk, D), lambda i: (i, 0)),
        out_shape=jax.ShapeDtypeStruct((BATCH, D), jnp.float32),
    )(xb, a0, a1, W2, b2.reshape(1, D))


def kernel(batch, edge_index, emb, W1, b1, W2, b2):
    a, bb = _precompute_ab(emb, W1, b1)
    xb, aggb = _sc_call(edge_index[0], edge_index[1], emb, a, bb, batch)
    return _final(xb, aggb[0], aggb[1], W2, b2)


# speedup vs baseline: 1.4679x; 1.0292x over previous
import functools

import jax
import jax.numpy as jnp
from jax import lax
from jax.experimental import pallas as pl
from jax.experimental.pallas import tpu as pltpu
from jax.experimental.pallas import tpu_sc as plsc

N_NODES = 10000
D = 128
E = 320000
BATCH = 4096

NC, NS = 2, 16
NW = NC * NS
EPW = E // NW
CK = 16
CKS = 4
NSETS = 16
NSETS_S = 4
SEGS = (5008, 4992)
BPW = BATCH // NW
BPT = BATCH // NS
GARBAGE = BATCH
ACC = 4112
ZR = ACC // NS
VL = 16
NMARK = N_NODES + VL


def _tc1_body(x_ref, w_ref, b_ref, a_ref, bb_ref):
    x = x_ref[...]
    a_ref[...] = jnp.dot(x, w_ref[:D, :], preferred_element_type=jnp.float32)
    bb_ref[...] = (
        jnp.dot(x, w_ref[D:, :], preferred_element_type=jnp.float32) + b_ref[...]
    )


def _precompute_ab(x, W1, b1):
    blk = N_NODES // 10
    return pl.pallas_call(
        _tc1_body,
        grid=(N_NODES // blk,),
        in_specs=[
            pl.BlockSpec((blk, D), lambda i: (i, 0)),
            pl.BlockSpec((2 * D, D), lambda i: (0, 0)),
            pl.BlockSpec((1, D), lambda i: (0, 0)),
        ],
        out_specs=[
            pl.BlockSpec((blk, D), lambda i: (i, 0)),
            pl.BlockSpec((blk, D), lambda i: (i, 0)),
        ],
        out_shape=[
            jax.ShapeDtypeStruct((N_NODES, D), jnp.float32),
            jax.ShapeDtypeStruct((N_NODES, D), jnp.float32),
        ],
    )(x, W1, b1.reshape(1, D))


def _sc_body(src_hbm, dst_hbm, emb_hbm, a_hbm, b_hbm, batch_hbm,
             xb_out, aggb_out, *scr):
    sets = tuple(
        tuple(scr[i * 5:(i + 1) * 5]) + (scr[5 * NSETS + 5 + i],)
        for i in range(NSETS)
    )
    bidxv, markv, srcsh, dstsh, aggsh = scr[5 * NSETS:5 * NSETS + 5]
    arows0 = sets[0][3]
    arows1 = sets[1][3]
    sg0 = sets[0][5]
    sg1 = sets[1][5]
    cid = lax.axis_index("c")
    sid = lax.axis_index("s")
    wid = sid * NC + cid

    zero16 = jnp.zeros((VL,), jnp.float32)
    ones16 = jnp.ones((VL,), jnp.int32)

    def zrow(i, carry):
        for j in range(D // VL):
            arows0[i, pl.ds(j * VL, VL)] = zero16
        return carry

    lax.fori_loop(0, CK, zrow, None)
    z0 = sid * ZR
    for r in range(ZR // CK):
        pltpu.sync_copy(arows0, aggsh.at[pl.ds(z0 + r * CK, CK)])
    pltpu.sync_copy(arows0.at[pl.ds(0, ZR - (ZR // CK) * CK)],
                    aggsh.at[pl.ds(z0 + (ZR // CK) * CK, ZR - (ZR // CK) * CK)])

    def zmark(i, carry):
        markv[pl.ds(i * VL, VL)] = jnp.zeros((VL,), jnp.int32)
        return carry

    lax.fori_loop(0, NMARK // VL, zmark, None)

    for ch in range(BATCH // BPW):
        pltpu.sync_copy(batch_hbm.at[pl.ds(ch * BPW, BPW)], bidxv)

        def scat(i, carry):
            idx = bidxv[pl.ds(i * VL, VL)]
            plsc.store_scatter(markv, [idx], ones16)
            return carry

        lax.fori_loop(0, BPW // VL, scat, None)

    def scan(i, carry):
        s = pl.ds(i * VL, VL)
        f = markv[s]
        ids = carry + plsc.cumsum(f) - 1
        markv[s] = jnp.where(f > 0, ids, jnp.full((VL,), GARBAGE, jnp.int32))
        return carry + jnp.sum(f)

    lax.fori_loop(0, NMARK // VL, scan, jnp.int32(0))

    plsc.subcore_barrier()

    pad_dst = jnp.full((VL,), N_NODES, jnp.int32)
    pad_src = jnp.zeros((VL,), jnp.int32)

    def run_segment(s0, selen):
        e0 = wid * EPW + s0
        pltpu.sync_copy(src_hbm.at[pl.ds(e0, selen)], srcsh.at[pl.ds(0, selen)])
        pltpu.sync_copy(dst_hbm.at[pl.ds(e0, selen)], dstsh.at[pl.ds(0, selen)])

        def compress(i, cur):
            s = pl.ds(i * VL, VL)
            d = dstsh[s]
            sv = srcsh[s]
            keep = plsc.load_gather(markv, [d]) != GARBAGE
            plsc.store_compressed(dstsh.at[pl.ds(cur, VL)], d, mask=keep)
            plsc.store_compressed(srcsh.at[pl.ds(cur, VL)], sv, mask=keep)
            return cur + jnp.sum(keep.astype(jnp.int32))

        cnt = lax.fori_loop(0, selen // VL, compress, jnp.int32(0))

        def padk(i, carry):
            dstsh[pl.ds(cnt + i * VL, VL)] = pad_dst
            srcsh[pl.ds(cnt + i * VL, VL)] = pad_src
            return carry

        lax.fori_loop(0, CK // VL, padk, None)
        nchunks = (cnt + CK - 1) >> CKS

        def build_issue(cc, p):
            sv, gv, cv, ar, br, sg = sets[p]
            for j in range(CK // VL):
                s = pl.ds(j * VL, VL)
                dv = dstsh[pl.ds(cc * CK + j * VL, VL)]
                sv[s] = srcsh[pl.ds(cc * CK + j * VL, VL)]
                gv[s] = jnp.minimum(dv, N_NODES - 1)
                cv[s] = plsc.load_gather(markv, [dv])
            pltpu.async_copy(a_hbm.at[sv], ar, sg)
            pltpu.async_copy(b_hbm.at[gv], br, sg)

        def step(c, p):
            sv, gv, cv, ar, br, sg = sets[p]
            pltpu.make_async_copy(a_hbm.at[sv], ar, sg).wait()
            pltpu.make_async_copy(b_hbm.at[gv], br, sg).wait()

            @pl.when(c + NSETS - 1 < nchunks)
            def _():
                build_issue(c + NSETS - 1, (p + NSETS - 1) % NSETS)

            def fuse(i, inner):
                for j in range(D // VL):
                    s = pl.ds(j * VL, VL)
                    ar[i, s] = jnp.maximum(ar[i, s] + br[i, s], 0.0)
                return inner

            lax.fori_loop(0, CK, fuse, None)
            pltpu.sync_copy(ar, aggsh.at[cv], add=True)

        for i in range(NSETS - 1):
            @pl.when(i < nchunks)
            def _(i=i):
                build_issue(i, i)

        def grp(cg, carry):
            c = cg * NSETS
            for b in range(NSETS):
                @pl.when(c + b < nchunks)
                def _(b=b):
                    step(c + b, b)
            return carry

        lax.fori_loop(0, (nchunks + NSETS - 1) >> NSETS_S, grp, None)

    s0 = 0
    for selen in SEGS:
        run_segment(s0, selen)
        s0 += selen
    plsc.subcore_barrier()

    base = wid * BPW
    pltpu.sync_copy(batch_hbm.at[pl.ds(base, BPW)], bidxv)
    for h in range(BPW // CK):
        pltpu.async_copy(emb_hbm.at[bidxv.at[pl.ds(h * CK, CK)]], arows0, sg0).wait()
        pltpu.sync_copy(arows0, xb_out.at[pl.ds(base + h * CK, CK)])

    for r in range(BPT // BPW):
        b0 = sid * BPT + r * BPW
        pltpu.sync_copy(batch_hbm.at[pl.ds(b0, BPW)], bidxv)
        for j in range(BPW // VL):
            s = pl.ds(j * VL, VL)
            bidxv[s] = plsc.load_gather(markv, [bidxv[s]])
        for h in range(BPW // CK):
            pltpu.async_copy(aggsh.at[bidxv.at[pl.ds(h * CK, CK)]], arows1, sg1).wait()
            pltpu.sync_copy(arows1, aggb_out.at[cid, pl.ds(b0 + h * CK, CK)])


def _set_scratch():
    return [
        pltpu.VMEM((CK,), jnp.int32),
        pltpu.VMEM((CK,), jnp.int32),
        pltpu.VMEM((CK,), jnp.int32),
        pltpu.VMEM((CK, D), jnp.float32),
        pltpu.VMEM((CK, D), jnp.float32),
    ]


_sc_call = pl.kernel(
    _sc_body,
    out_type=(
        jax.ShapeDtypeStruct((BATCH, D), jnp.float32),
        jax.ShapeDtypeStruct((NC, BATCH, D), jnp.float32),
    ),
    mesh=plsc.VectorSubcoreMesh(core_axis_name="c", subcore_axis_name="s"),
    scratch_types=(
        sum((_set_scratch() for _ in range(NSETS)), []) + [
            pltpu.VMEM((BPW,), jnp.int32),
            pltpu.VMEM((NMARK,), jnp.int32),
            pltpu.VMEM((SEGS[0] + CK,), jnp.int32),
            pltpu.VMEM((SEGS[0] + CK,), jnp.int32),
            pltpu.VMEM_SHARED((ACC, D), jnp.float32),
        ] + [pltpu.SemaphoreType.DMA] * NSETS
    ),
    compiler_params=pltpu.CompilerParams(needs_layout_passes=False),
)


def _tc2_body(xb_ref, a0_ref, a1_ref, w_ref, b_ref, o_ref):
    o_ref[...] = (
        jnp.dot(xb_ref[...], w_ref[:D, :], preferred_element_type=jnp.float32)
        + jnp.dot(a0_ref[...] + a1_ref[...], w_ref[D:, :],
                  preferred_element_type=jnp.float32)
        + b_ref[...]
    )


def _final(xb, a0, a1, W2, b2):
    blk = 1024
    return pl.pallas_call(
        _tc2_body,
        grid=(BATCH // blk,),
        in_specs=[
            pl.BlockSpec((blk, D), lambda i: (i, 0)),
            pl.BlockSpec((blk, D), lambda i: (i, 0)),
            pl.BlockSpec((blk, D), lambda i: (i, 0)),
            pl.BlockSpec((2 * D, D), lambda i: (0, 0)),
            pl.BlockSpec((1, D), lambda i: (0, 0)),
        ],
        out_specs=pl.BlockSpec((blk, D), lambda i: (i, 0)),
        out_shape=jax.ShapeDtypeStruct((BATCH, D), jnp.float32),
    )(xb, a0, a1, W2, b2.reshape(1, D))


def kernel(batch, edge_index, emb, W1, b1, W2, b2):
    a, bb = _precompute_ab(emb, W1, b1)
    xb, aggb = _sc_call(edge_index[0], edge_index[1], emb, a, bb, batch)
    return _final(xb, aggb[0], aggb[1], W2, b2)


# speedup vs baseline: 1.5064x; 1.0262x over previous
import functools

import jax
import jax.numpy as jnp
from jax import lax
from jax.experimental import pallas as pl
from jax.experimental.pallas import tpu as pltpu
from jax.experimental.pallas import tpu_sc as plsc

N_NODES = 10000
D = 128
E = 320000
BATCH = 4096

NC, NS = 2, 16
NW = NC * NS
EPW = E // NW
CK = 16
CKS = 4
NSETS = 16
NSETS_S = 4
SEGS = (5008, 4992)
BPW = BATCH // NW
BPT = BATCH // NS
GARBAGE = BATCH
ACC = 4112
ZR = ACC // NS
VL = 16
NMARK = N_NODES + VL


def _tc1_body(x_ref, w_ref, b_ref, a_ref, bb_ref):
    x = x_ref[...]
    a_ref[...] = jnp.dot(x, w_ref[:D, :], preferred_element_type=jnp.float32)
    bb_ref[...] = (
        jnp.dot(x, w_ref[D:, :], preferred_element_type=jnp.float32) + b_ref[...]
    )


def _precompute_ab(x, W1, b1):
    blk = N_NODES // 10
    return pl.pallas_call(
        _tc1_body,
        grid=(N_NODES // blk,),
        in_specs=[
            pl.BlockSpec((blk, D), lambda i: (i, 0)),
            pl.BlockSpec((2 * D, D), lambda i: (0, 0)),
            pl.BlockSpec((1, D), lambda i: (0, 0)),
        ],
        out_specs=[
            pl.BlockSpec((blk, D), lambda i: (i, 0)),
            pl.BlockSpec((blk, D), lambda i: (i, 0)),
        ],
        out_shape=[
            jax.ShapeDtypeStruct((N_NODES, D), jnp.float32),
            jax.ShapeDtypeStruct((N_NODES, D), jnp.float32),
        ],
    )(x, W1, b1.reshape(1, D))


def _sc_body(src_hbm, dst_hbm, emb_hbm, a_hbm, b_hbm, batch_hbm,
             xb_out, aggb_out, *scr):
    sets = tuple(
        tuple(scr[i * 5:(i + 1) * 5]) + (scr[5 * NSETS + 5 + i],)
        for i in range(NSETS)
    )
    bidxv, markv, srcsh, dstsh, aggsh = scr[5 * NSETS:5 * NSETS + 5]
    arows0 = sets[0][3]
    arows1 = sets[1][3]
    sg0 = sets[0][5]
    sg1 = sets[1][5]
    cid = lax.axis_index("c")
    sid = lax.axis_index("s")
    wid = sid * NC + cid

    zero16 = jnp.zeros((VL,), jnp.float32)
    ones16 = jnp.ones((VL,), jnp.int32)

    def zrow(i, carry):
        for j in range(D // VL):
            arows0[i, pl.ds(j * VL, VL)] = zero16
        return carry

    lax.fori_loop(0, CK, zrow, None)
    z0 = sid * ZR
    for r in range(ZR // CK):
        pltpu.sync_copy(arows0, aggsh.at[pl.ds(z0 + r * CK, CK)])
    pltpu.sync_copy(arows0.at[pl.ds(0, ZR - (ZR // CK) * CK)],
                    aggsh.at[pl.ds(z0 + (ZR // CK) * CK, ZR - (ZR // CK) * CK)])

    def zmark(i, carry):
        markv[pl.ds(i * VL, VL)] = jnp.zeros((VL,), jnp.int32)
        return carry

    lax.fori_loop(0, NMARK // VL, zmark, None)

    for ch in range(BATCH // BPW):
        pltpu.sync_copy(batch_hbm.at[pl.ds(ch * BPW, BPW)], bidxv)

        def scat(i, carry):
            idx = bidxv[pl.ds(i * VL, VL)]
            plsc.store_scatter(markv, [idx], ones16)
            return carry

        lax.fori_loop(0, BPW // VL, scat, None)

    def scan(i, carry):
        s = pl.ds(i * VL, VL)
        f = markv[s]
        ids = carry + plsc.cumsum(f) - 1
        markv[s] = jnp.where(f > 0, ids, jnp.full((VL,), GARBAGE, jnp.int32))
        return carry + jnp.sum(f)

    lax.fori_loop(0, NMARK // VL, scan, jnp.int32(0))

    plsc.subcore_barrier()

    pad_dst = jnp.full((VL,), N_NODES, jnp.int32)
    pad_src = jnp.zeros((VL,), jnp.int32)

    def run_segment(s0, selen):
        e0 = wid * EPW + s0
        pltpu.sync_copy(src_hbm.at[pl.ds(e0, selen)], srcsh.at[pl.ds(0, selen)])
        pltpu.sync_copy(dst_hbm.at[pl.ds(e0, selen)], dstsh.at[pl.ds(0, selen)])

        def compress(i, cur):
            s = pl.ds(i * VL, VL)
            d = dstsh[s]
            sv = srcsh[s]
            keep = plsc.load_gather(markv, [d]) != GARBAGE
            plsc.store_compressed(dstsh.at[pl.ds(cur, VL)], d, mask=keep)
            plsc.store_compressed(srcsh.at[pl.ds(cur, VL)], sv, mask=keep)
            return cur + jnp.sum(keep.astype(jnp.int32))

        cnt = lax.fori_loop(0, selen // VL, compress, jnp.int32(0))

        def padk(i, carry):
            dstsh[pl.ds(cnt + i * VL, VL)] = pad_dst
            srcsh[pl.ds(cnt + i * VL, VL)] = pad_src
            return carry

        lax.fori_loop(0, CK // VL, padk, None)
        nchunks = (cnt + CK - 1) >> CKS

        def build_issue(cc, p):
            sv, gv, cv, ar, br, sg = sets[p]
            for j in range(CK // VL):
                s = pl.ds(j * VL, VL)
                dv = dstsh[pl.ds(cc * CK + j * VL, VL)]
                sv[s] = srcsh[pl.ds(cc * CK + j * VL, VL)]
                gv[s] = jnp.minimum(dv, N_NODES - 1)
                cv[s] = plsc.load_gather(markv, [dv])
            pltpu.async_copy(a_hbm.at[sv], ar, sg)
            pltpu.async_copy(b_hbm.at[gv], br, sg)

        def step(c, p):
            sv, gv, cv, ar, br, sg = sets[p]
            pltpu.make_async_copy(a_hbm.at[sv], ar, sg).wait()
            pltpu.make_async_copy(b_hbm.at[gv], br, sg).wait()

            @pl.when(c + NSETS - 1 < nchunks)
            def _():
                build_issue(c + NSETS - 1, (p + NSETS - 1) % NSETS)

            def fuse(i, inner):
                for j in range(D // VL):
                    s = pl.ds(j * VL, VL)
                    ar[i, s] = jnp.maximum(ar[i, s] + br[i, s], 0.0)
                return inner

            lax.fori_loop(0, CK, fuse, None)
            pltpu.sync_copy(ar, aggsh.at[cv], add=True)

        for i in range(NSETS - 1):
            @pl.when(i < nchunks)
            def _(i=i):
                build_issue(i, i)

        def grp(cg, carry):
            c = cg * NSETS
            for b in range(NSETS):
                @pl.when(c + b < nchunks)
                def _(b=b):
                    step(c + b, b)
            return carry

        lax.fori_loop(0, (nchunks + NSETS - 1) >> NSETS_S, grp, None)

    s0 = 0
    for selen in SEGS:
        run_segment(s0, selen)
        s0 += selen
    plsc.subcore_barrier()

    base = wid * BPW
    pltpu.sync_copy(batch_hbm.at[pl.ds(base, BPW)], bidxv)
    for h in range(BPW // CK):
        pltpu.async_copy(emb_hbm.at[bidxv.at[pl.ds(h * CK, CK)]], arows0, sg0).wait()
        pltpu.sync_copy(arows0, xb_out.at[pl.ds(base + h * CK, CK)])

    for r in range(BPT // BPW):
        b0 = sid * BPT + r * BPW
        pltpu.sync_copy(batch_hbm.at[pl.ds(b0, BPW)], bidxv)
        for j in range(BPW // VL):
            s = pl.ds(j * VL, VL)
            bidxv[s] = plsc.load_gather(markv, [bidxv[s]])
        for h in range(BPW // CK):
            pltpu.async_copy(aggsh.at[bidxv.at[pl.ds(h * CK, CK)]], arows1, sg1).wait()
            pltpu.sync_copy(arows1, aggb_out.at[cid, pl.ds(b0 + h * CK, CK)])


def _set_scratch():
    return [
        pltpu.VMEM((CK,), jnp.int32),
        pltpu.VMEM((CK,), jnp.int32),
        pltpu.VMEM((CK,), jnp.int32),
        pltpu.VMEM((CK, D), jnp.float32),
        pltpu.VMEM((CK, D), jnp.float32),
    ]


_sc_call = pl.kernel(
    _sc_body,
    out_type=(
        jax.ShapeDtypeStruct((BATCH, D), jnp.float32),
        jax.ShapeDtypeStruct((NC, BATCH, D), jnp.float32),
    ),
    mesh=plsc.VectorSubcoreMesh(core_axis_name="c", subcore_axis_name="s"),
    scratch_types=(
        sum((_set_scratch() for _ in range(NSETS)), []) + [
            pltpu.VMEM((BPW,), jnp.int32),
            pltpu.VMEM((NMARK,), jnp.int32),
            pltpu.VMEM((SEGS[0] + CK,), jnp.int32),
            pltpu.VMEM((SEGS[0] + CK,), jnp.int32),
            pltpu.VMEM_SHARED((ACC, D), jnp.float32),
        ] + [pltpu.SemaphoreType.DMA] * NSETS
    ),
    compiler_params=pltpu.CompilerParams(needs_layout_passes=False),
)


def _tc2_body(xb_ref, a0_ref, a1_ref, w_ref, b_ref, o_ref):
    o_ref[...] = (
        jnp.dot(xb_ref[...], w_ref[:D, :], preferred_element_type=jnp.float32)
        + jnp.dot(a0_ref[0] + a1_ref[0], w_ref[D:, :],
                  preferred_element_type=jnp.float32)
        + b_ref[...]
    )


def _final(xb, aggb, W2, b2):
    blk = 1024
    return pl.pallas_call(
        _tc2_body,
        grid=(BATCH // blk,),
        in_specs=[
            pl.BlockSpec((blk, D), lambda i: (i, 0)),
            pl.BlockSpec((1, blk, D), lambda i: (0, i, 0)),
            pl.BlockSpec((1, blk, D), lambda i: (1, i, 0)),
            pl.BlockSpec((2 * D, D), lambda i: (0, 0)),
            pl.BlockSpec((1, D), lambda i: (0, 0)),
        ],
        out_specs=pl.BlockSpec((blk, D), lambda i: (i, 0)),
        out_shape=jax.ShapeDtypeStruct((BATCH, D), jnp.float32),
    )(xb, aggb, aggb, W2, b2.reshape(1, D))


def kernel(batch, edge_index, emb, W1, b1, W2, b2):
    a, bb = _precompute_ab(emb, W1, b1)
    xb, aggb = _sc_call(edge_index[0], edge_index[1], emb, a, bb, batch)
    return _final(xb, aggb, W2, b2)
